# baseline (speedup 1.0000x reference)
import functools

import jax
import jax.numpy as jnp
from jax import lax
from jax.experimental import pallas as pl
from jax.experimental.pallas import tpu as pltpu
from jax.experimental.pallas import tpu_sc as plsc

_N_E = 8192
_D = 32
_N = 8192
_RB = 512
_KB = 512


def _argmin_body(z_ref, a_ref, cbm2_ref, c_ref, idx_ref, dmin_ref):
    z = z_ref[0]
    a = a_ref[...]
    run_min = jnp.full((1, _RB), jnp.inf, dtype=jnp.float32)
    run_idx = jnp.zeros((1, _RB), dtype=jnp.float32)
    iota = lax.broadcasted_iota(jnp.int32, (_KB, _RB), 0).astype(jnp.float32)

    kc = _N_E // _KB

    def _mm(k):
        e = cbm2_ref[pl.ds(k * _KB, _KB), :]
        return lax.dot_general(e, z, (((1,), (0,)), ((), ())),
                               preferred_element_type=jnp.float32)

    def chunk(k, carry):
        run_min, run_idx, mm = carry
        mm_next = _mm(lax.min(k + 1, kc - 1))
        c = c_ref[pl.ds(k * _KB, _KB), :]
        d = (a + mm) + c
        m = jnp.min(d, axis=0, keepdims=True)
        li = jnp.min(jnp.where(d == m, iota, 3e38), axis=0, keepdims=True)
        better = m < run_min
        run_idx = jnp.where(better, li + jnp.float32(k * _KB), run_idx)
        run_min = jnp.where(better, m, run_min)
        return run_min, run_idx, mm_next

    run_min, run_idx, _ = lax.fori_loop(0, kc, chunk,
                                        (run_min, run_idx, _mm(0)), unroll=2)
    idx_ref[...] = run_idx.astype(jnp.int32).reshape(1, 1, _RB)
    dmin_ref[...] = run_min.reshape(1, 1, _RB)


def _nearest_codes(z3, a_t, cbm2, c_t):
    nb = _N // _RB
    hw = z3.shape[2]
    per_b = hw // _RB if hw >= _RB else 1
    return pl.pallas_call(
        _argmin_body,
        grid=(nb,),
        in_specs=[
            pl.BlockSpec((1, _D, _RB), lambda i: (i // 2, 0, i % 2)),
            pl.BlockSpec((1, _RB), lambda i: (0, i)),
            pl.BlockSpec((_N_E, _D), lambda i: (0, 0)),
            pl.BlockSpec((_N_E, 1), lambda i: (0, 0)),
        ],
        out_specs=[
            pl.BlockSpec((1, 1, _RB), lambda i: (i, 0, 0)),
            pl.BlockSpec((1, 1, _RB), lambda i: (i, 0, 0)),
        ],
        out_shape=[
            jax.ShapeDtypeStruct((nb, 1, _RB), jnp.int32),
            jax.ShapeDtypeStruct((nb, 1, _RB), jnp.float32),
        ],
    )(z3, a_t, cbm2, c_t)


def _sc_gather(cb, idx2d):
    info = plsc.get_sparse_core_info()
    nc, ns = info.num_cores, info.num_subcores
    nw = nc * ns
    rows_w = _N // nw
    jn = rows_w // 128
    mesh = plsc.VectorSubcoreMesh(core_axis_name="c", subcore_axis_name="s")

    @functools.partial(
        pl.kernel, mesh=mesh,
        out_type=jax.ShapeDtypeStruct((_N, _D), jnp.float32),
        compiler_params=pltpu.CompilerParams(use_tc_tiling_on_sc=False),
        scratch_types=[
            pltpu.VMEM((jn, 128), jnp.int32),
            pltpu.VMEM((rows_w, _D), jnp.float32),
            pltpu.SemaphoreType.DMA,
        ],
    )
    def gather_k(cb_hbm, idx_hbm, out_hbm, idx_v, rows_v, sem):
        wid = lax.axis_index("s") * nc + lax.axis_index("c")
        pltpu.sync_copy(idx_hbm.at[pl.ds(wid * jn, jn)], idx_v)
        copies = [
            pltpu.async_copy(cb_hbm.at[idx_v.at[j]],
                             rows_v.at[pl.ds(j * 128, 128)], sem)
            for j in range(jn)
        ]
        for cp in copies:
            cp.wait()
        pltpu.sync_copy(rows_v, out_hbm.at[pl.ds(wid * rows_w, rows_w)])

    return gather_k(cb, idx2d)


def kernel(z, codebook):
    bz, ch, hh, ww = z.shape
    z3 = z.reshape(bz, ch, hh * ww)
    a = jnp.sum(z3 ** 2, axis=1)
    c = jnp.sum(codebook.T ** 2, axis=0)

    idx_b, dmin_b = _nearest_codes(z3, a.reshape(1, _N), -2.0 * codebook,
                                   c.reshape(_N_E, 1))
    idx_flat = idx_b.reshape(_N)

    zq_flat = _sc_gather(codebook, idx_flat.reshape(_N // 128, 128))

    zq_t = jnp.transpose(zq_flat.reshape(bz, hh, ww, ch), (0, 3, 1, 2))
    loss = 2.0 * (jnp.sum(dmin_b) / (1.0 * z.size))
    z_q = z + (zq_t - z)
    return (z_q, jnp.float32(loss), idx_b.reshape(bz, hh * ww))


# speedup vs baseline: 1.2765x; 1.2765x over previous
import functools

import jax
import jax.numpy as jnp
from jax import lax
from jax.experimental import pallas as pl
from jax.experimental.pallas import tpu as pltpu
from jax.experimental.pallas import tpu_sc as plsc

_N_E = 8192
_D = 32
_N = 8192
_RB = 512
_KB = 512


def _argmin_body(z_ref, a_ref, cbm2_ref, c_ref, idx_ref, dmin_ref):
    z = z_ref[0]
    a = a_ref[...]
    run_min = jnp.full((1, _RB), jnp.inf, dtype=jnp.float32)
    run_idx = jnp.zeros((1, _RB), dtype=jnp.float32)
    iota = lax.broadcasted_iota(jnp.int32, (_KB, _RB), 0).astype(jnp.float32)

    kc = _N_E // _KB
    for k in range(kc):
        e = cbm2_ref[pl.ds(k * _KB, _KB), :]
        mm = lax.dot_general(e, z, (((1,), (0,)), ((), ())),
                             preferred_element_type=jnp.float32)
        c = c_ref[pl.ds(k * _KB, _KB), :]
        d = (a + mm) + c
        m = jnp.min(d, axis=0, keepdims=True)
        li = jnp.min(jnp.where(d == m, iota, 3e38), axis=0, keepdims=True)
        better = m < run_min
        run_idx = jnp.where(better, li + jnp.float32(k * _KB), run_idx)
        run_min = jnp.where(better, m, run_min)
    idx_ref[...] = run_idx.astype(jnp.int32).reshape(1, 1, _RB)
    dmin_ref[...] = run_min.reshape(1, 1, _RB)


def _nearest_codes(z3, a_t, cbm2, c_t):
    nb = _N // _RB
    hw = z3.shape[2]
    per_b = hw // _RB if hw >= _RB else 1
    return pl.pallas_call(
        _argmin_body,
        grid=(nb,),
        in_specs=[
            pl.BlockSpec((1, _D, _RB), lambda i: (i // 2, 0, i % 2)),
            pl.BlockSpec((1, _RB), lambda i: (0, i)),
            pl.BlockSpec((_N_E, _D), lambda i: (0, 0)),
            pl.BlockSpec((_N_E, 1), lambda i: (0, 0)),
        ],
        out_specs=[
            pl.BlockSpec((1, 1, _RB), lambda i: (i, 0, 0)),
            pl.BlockSpec((1, 1, _RB), lambda i: (i, 0, 0)),
        ],
        out_shape=[
            jax.ShapeDtypeStruct((nb, 1, _RB), jnp.int32),
            jax.ShapeDtypeStruct((nb, 1, _RB), jnp.float32),
        ],
    )(z3, a_t, cbm2, c_t)


def _sc_gather(cb, idx2d):
    info = plsc.get_sparse_core_info()
    nc, ns = info.num_cores, info.num_subcores
    nw = nc * ns
    rows_w = _N // nw
    jn = rows_w // 128
    mesh = plsc.VectorSubcoreMesh(core_axis_name="c", subcore_axis_name="s")

    @functools.partial(
        pl.kernel, mesh=mesh,
        out_type=jax.ShapeDtypeStruct((_N, _D), jnp.float32),
        compiler_params=pltpu.CompilerParams(use_tc_tiling_on_sc=False),
        scratch_types=[
            pltpu.VMEM((jn, 128), jnp.int32),
            pltpu.VMEM((rows_w, _D), jnp.float32),
            pltpu.SemaphoreType.DMA,
        ],
    )
    def gather_k(cb_hbm, idx_hbm, out_hbm, idx_v, rows_v, sem):
        wid = lax.axis_index("s") * nc + lax.axis_index("c")
        pltpu.sync_copy(idx_hbm.at[pl.ds(wid * jn, jn)], idx_v)
        copies = [
            pltpu.async_copy(cb_hbm.at[idx_v.at[j]],
                             rows_v.at[pl.ds(j * 128, 128)], sem)
            for j in range(jn)
        ]
        for cp in copies:
            cp.wait()
        pltpu.sync_copy(rows_v, out_hbm.at[pl.ds(wid * rows_w, rows_w)])

    return gather_k(cb, idx2d)


def kernel(z, codebook):
    bz, ch, hh, ww = z.shape
    z3 = z.reshape(bz, ch, hh * ww)
    a = jnp.sum(z3 ** 2, axis=1)
    c = jnp.sum(codebook.T ** 2, axis=0)

    idx_b, dmin_b = _nearest_codes(z3, a.reshape(1, _N), -2.0 * codebook,
                                   c.reshape(_N_E, 1))
    idx_flat = idx_b.reshape(_N)

    zq_flat = _sc_gather(codebook, idx_flat.reshape(_N // 128, 128))

    zq_t = jnp.transpose(zq_flat.reshape(bz, hh, ww, ch), (0, 3, 1, 2))
    loss = 2.0 * (jnp.sum(dmin_b) / (1.0 * z.size))
    z_q = z + (zq_t - z)
    return (z_q, jnp.float32(loss), idx_b.reshape(bz, hh * ww))


# speedup vs baseline: 1.3186x; 1.0330x over previous
import functools

import jax
import jax.numpy as jnp
from jax import lax
from jax.experimental import pallas as pl
from jax.experimental.pallas import tpu as pltpu
from jax.experimental.pallas import tpu_sc as plsc

_N_E = 8192
_D = 32
_N = 8192
_RB = 512
_KB = 512


def _argmin_body(z_ref, a_ref, cb_ref, c_ref, idx_ref, dmin_ref):
    zm2 = -2.0 * z_ref[0]
    a = a_ref[...]
    run_min = jnp.full((1, _RB), jnp.inf, dtype=jnp.float32)
    run_idx = jnp.zeros((1, _RB), dtype=jnp.float32)
    iota = lax.broadcasted_iota(jnp.int32, (_KB, _RB), 0).astype(jnp.float32)

    kc = _N_E // _KB
    for k in range(kc):
        e = cb_ref[pl.ds(k * _KB, _KB), :]
        mm = lax.dot_general(e, zm2, (((1,), (0,)), ((), ())),
                             preferred_element_type=jnp.float32)
        c = c_ref[pl.ds(k * _KB, _KB), :]
        d = (a + mm) + c
        m = jnp.min(d, axis=0, keepdims=True)
        li = jnp.min(jnp.where(d == m, iota, 3e38), axis=0, keepdims=True)
        better = m < run_min
        run_idx = jnp.where(better, li + jnp.float32(k * _KB), run_idx)
        run_min = jnp.where(better, m, run_min)

    idx_ref[...] = run_idx.astype(jnp.int32).reshape(1, 1, _RB)
    dmin_ref[...] = run_min.reshape(1, 1, _RB)


def _nearest_codes(z3, a_t, cb, c_t):
    nb = _N // _RB
    return pl.pallas_call(
        _argmin_body,
        grid=(nb,),
        in_specs=[
            pl.BlockSpec((1, _D, _RB), lambda i: (i // 2, 0, i % 2)),
            pl.BlockSpec((1, _RB), lambda i: (0, i)),
            pl.BlockSpec((_N_E, _D), lambda i: (0, 0)),
            pl.BlockSpec((_N_E, 1), lambda i: (0, 0)),
        ],
        out_specs=[
            pl.BlockSpec((1, 1, _RB), lambda i: (i, 0, 0)),
            pl.BlockSpec((1, 1, _RB), lambda i: (i, 0, 0)),
        ],
        out_shape=[
            jax.ShapeDtypeStruct((nb, 1, _RB), jnp.int32),
            jax.ShapeDtypeStruct((nb, 1, _RB), jnp.float32),
        ],
    )(z3, a_t, cb, c_t)


def _sc_gather(cb, idx2d):
    info = plsc.get_sparse_core_info()
    nc, ns = info.num_cores, info.num_subcores
    nw = nc * ns
    rows_w = _N // nw
    jn = rows_w // 128
    mesh = plsc.VectorSubcoreMesh(core_axis_name="c", subcore_axis_name="s")

    @functools.partial(
        pl.kernel, mesh=mesh,
        out_type=jax.ShapeDtypeStruct((_N, _D), jnp.float32),
        compiler_params=pltpu.CompilerParams(use_tc_tiling_on_sc=False),
        scratch_types=[
            pltpu.VMEM((jn, 128), jnp.int32),
            pltpu.VMEM((rows_w, _D), jnp.float32),
            pltpu.SemaphoreType.DMA,
        ],
    )
    def gather_k(cb_hbm, idx_hbm, out_hbm, idx_v, rows_v, sem):
        wid = lax.axis_index("s") * nc + lax.axis_index("c")
        pltpu.sync_copy(idx_hbm.at[pl.ds(wid * jn, jn)], idx_v)
        copies = [
            pltpu.async_copy(cb_hbm.at[idx_v.at[j]],
                             rows_v.at[pl.ds(j * 128, 128)], sem)
            for j in range(jn)
        ]
        for cp in copies:
            cp.wait()
        pltpu.sync_copy(rows_v, out_hbm.at[pl.ds(wid * rows_w, rows_w)])

    return gather_k(cb, idx2d)


def _ste_body(zq_ref, z_ref, out_ref):
    zq_t = zq_ref[...].T
    z = z_ref[0]
    out_ref[0, :, :] = z + (zq_t - z)


def _ste_transpose(zq_flat, z3):
    nb = _N // _RB
    return pl.pallas_call(
        _ste_body,
        grid=(nb,),
        in_specs=[
            pl.BlockSpec((_RB, _D), lambda i: (i, 0)),
            pl.BlockSpec((1, _D, _RB), lambda i: (i // 2, 0, i % 2)),
        ],
        out_specs=pl.BlockSpec((1, _D, _RB), lambda i: (i // 2, 0, i % 2)),
        out_shape=jax.ShapeDtypeStruct(z3.shape, jnp.float32),
    )(zq_flat, z3)


def kernel(z, codebook):
    bz, ch, hh, ww = z.shape
    z3 = z.reshape(bz, ch, hh * ww)
    a = jnp.sum(z3 ** 2, axis=1)
    c = jnp.sum(codebook.T ** 2, axis=0)

    idx_b, dmin_b = _nearest_codes(z3, a.reshape(1, _N), codebook,
                                   c.reshape(_N_E, 1))

    zq_flat = _sc_gather(codebook, idx_b.reshape(_N // 128, 128))

    z_q = _ste_transpose(zq_flat, z3).reshape(z.shape)
    loss = 2.0 * (jnp.sum(dmin_b) / (1.0 * z.size))
    return (z_q, jnp.float32(loss), idx_b.reshape(bz, hh * ww))


# speedup vs baseline: 1.3945x; 1.0576x over previous
import functools

import jax
import jax.numpy as jnp
from jax import lax
from jax.experimental import pallas as pl
from jax.experimental.pallas import tpu as pltpu
from jax.experimental.pallas import tpu_sc as plsc

_N_E = 8192
_D = 32
_N = 8192
_RB = 1024
_KB = 512


def _argmin_body(z_ref, a_ref, cb_ref, c_ref, idx_ref, dmin_ref):
    zm2 = -2.0 * z_ref[0]
    a = a_ref[...]
    run_min = jnp.full((1, _RB), jnp.inf, dtype=jnp.float32)
    run_idx = jnp.zeros((1, _RB), dtype=jnp.float32)
    iota = lax.broadcasted_iota(jnp.int32, (_KB, _RB), 0).astype(jnp.float32)

    kc = _N_E // _KB
    for k in range(kc):
        e = cb_ref[pl.ds(k * _KB, _KB), :]
        mm = lax.dot_general(e, zm2, (((1,), (0,)), ((), ())),
                             preferred_element_type=jnp.float32)
        c = c_ref[pl.ds(k * _KB, _KB), :]
        d = (a + mm) + c
        m = jnp.min(d, axis=0, keepdims=True)
        li = jnp.min(jnp.where(d == m, iota, 3e38), axis=0, keepdims=True)
        better = m < run_min
        run_idx = jnp.where(better, li + jnp.float32(k * _KB), run_idx)
        run_min = jnp.where(better, m, run_min)

    idx_ref[...] = run_idx.astype(jnp.int32).reshape(1, 1, _RB)
    dmin_ref[...] = run_min.reshape(1, 1, _RB)


def _nearest_codes(z3, a_t, cb, c_t):
    nb = _N // _RB
    return pl.pallas_call(
        _argmin_body,
        grid=(nb,),
        in_specs=[
            pl.BlockSpec((1, _D, _RB), lambda i: (i, 0, 0)),
            pl.BlockSpec((1, _RB), lambda i: (0, i)),
            pl.BlockSpec((_N_E, _D), lambda i: (0, 0)),
            pl.BlockSpec((_N_E, 1), lambda i: (0, 0)),
        ],
        out_specs=[
            pl.BlockSpec((1, 1, _RB), lambda i: (i, 0, 0)),
            pl.BlockSpec((1, 1, _RB), lambda i: (i, 0, 0)),
        ],
        out_shape=[
            jax.ShapeDtypeStruct((nb, 1, _RB), jnp.int32),
            jax.ShapeDtypeStruct((nb, 1, _RB), jnp.float32),
        ],
    )(z3, a_t, cb, c_t)


def _sc_gather(cb, idx2d):
    info = plsc.get_sparse_core_info()
    nc, ns = info.num_cores, info.num_subcores
    nw = nc * ns
    rows_w = _N // nw
    jn = rows_w // 128
    mesh = plsc.VectorSubcoreMesh(core_axis_name="c", subcore_axis_name="s")

    @functools.partial(
        pl.kernel, mesh=mesh,
        out_type=jax.ShapeDtypeStruct((_N, _D), jnp.float32),
        compiler_params=pltpu.CompilerParams(use_tc_tiling_on_sc=False),
        scratch_types=[
            pltpu.VMEM((jn, 128), jnp.int32),
            pltpu.VMEM((rows_w, _D), jnp.float32),
            pltpu.SemaphoreType.DMA,
        ],
    )
    def gather_k(cb_hbm, idx_hbm, out_hbm, idx_v, rows_v, sem):
        wid = lax.axis_index("s") * nc + lax.axis_index("c")
        pltpu.sync_copy(idx_hbm.at[pl.ds(wid * jn, jn)], idx_v)
        copies = [
            pltpu.async_copy(cb_hbm.at[idx_v.at[j]],
                             rows_v.at[pl.ds(j * 128, 128)], sem)
            for j in range(jn)
        ]
        for cp in copies:
            cp.wait()
        pltpu.sync_copy(rows_v, out_hbm.at[pl.ds(wid * rows_w, rows_w)])

    return gather_k(cb, idx2d)


def _ste_body(zq_ref, z_ref, out_ref):
    nb = z_ref.shape[0]
    hw = z_ref.shape[2]
    zq_t = zq_ref[...].T
    for b in range(nb):
        z = z_ref[b]
        out_ref[b, :, :] = z + (zq_t[:, b * hw:(b + 1) * hw] - z)


def _ste_transpose(zq_flat, z3):
    bh = 4
    return pl.pallas_call(
        _ste_body,
        grid=(z3.shape[0] // bh,),
        in_specs=[
            pl.BlockSpec((bh * 1024, _D), lambda i: (i, 0)),
            pl.BlockSpec((bh, _D, 1024), lambda i: (i, 0, 0)),
        ],
        out_specs=pl.BlockSpec((bh, _D, 1024), lambda i: (i, 0, 0)),
        out_shape=jax.ShapeDtypeStruct(z3.shape, jnp.float32),
    )(zq_flat, z3)


def kernel(z, codebook):
    bz, ch, hh, ww = z.shape
    z3 = z.reshape(bz, ch, hh * ww)
    a = jnp.sum(z3 ** 2, axis=1)
    c = jnp.sum(codebook.T ** 2, axis=0)

    idx_b, dmin_b = _nearest_codes(z3, a.reshape(1, _N), codebook,
                                   c.reshape(_N_E, 1))

    zq_flat = _sc_gather(codebook, idx_b.reshape(_N // 128, 128))

    z_q = _ste_transpose(zq_flat, z3).reshape(z.shape)
    loss = 2.0 * (jnp.sum(dmin_b) / (1.0 * z.size))
    return (z_q, jnp.float32(loss), idx_b.reshape(bz, hh * ww))
